# Initial kernel scaffold; baseline (speedup 1.0000x reference)
import jax
import jax.numpy as jnp
from jax.experimental import pallas as pl


def kernel(task_state_scheduled, task_state_ready, task_completion_time, vm_completion_time, adj, gin0_W1, gin0_b1, gin0_gamma, gin0_beta, gin0_W2, gin0_b2, gin1_W1, gin1_b1, gin1_gamma, gin1_beta, gin1_W2, gin1_b2, gin2_W1, gin2_b1, gin2_gamma, gin2_beta, gin2_W2, gin2_b2, me_W1, me_b1, me_W2, me_b2, dec_W1, dec_b1, dec_W2, dec_b2, dec_W3, dec_b3):
    raise NotImplementedError("write your pallas kernel here")


# speedup vs baseline: 822.0354x; 822.0354x over previous
import jax
import jax.numpy as jnp
from jax.experimental import pallas as pl

N_JOBS = 1024
N_MACHINES = 64
H = 128

_F32 = jnp.float32
_BF16 = jnp.bfloat16
_HI = jax.lax.Precision.HIGHEST


def _bdot(a, b):
    return jnp.dot(a.astype(_BF16), b.astype(_BF16),
                   preferred_element_type=_F32)


def _fused_forward(feats, ready, vm, adj,
                   g0W1, g0b1, g0g, g0bt, g0W2, g0b2,
                   g1W1, g1b1, g1g, g1bt, g1W2, g1b2,
                   g2W1, g2b1, g2g, g2bt, g2W2, g2b2,
                   meW1, meb1, meW2, meb2,
                   dW1, db1, dW2, db2, dW3, db3,
                   probs_out, graph_out, mach_out):
    a = adj[:]
    x = feats[:]

    def gin_layer(x, W1, b1, gamma, beta, W2, b2, last):
        agg = jax.lax.dot_general(
            a, x, (((0,), (0,)), ((), ())),
            preferred_element_type=_F32, precision=_HI)
        h = _bdot(x + agg, W1[:]) + b1[:]
        mu = jnp.mean(h, axis=0, keepdims=True)
        var = jnp.mean((h - mu) * (h - mu), axis=0, keepdims=True)
        h = (h - mu) * jax.lax.rsqrt(var + 1e-5) * gamma[:] + beta[:]
        h = jnp.maximum(h, 0.0)
        h = _bdot(h, W2[:]) + b2[:]
        if not last:
            h = jnp.maximum(h, 0.0)
        return h

    x = gin_layer(x, g0W1, g0b1, g0g, g0bt, g0W2, g0b2, last=False)
    x = gin_layer(x, g1W1, g1b1, g1g, g1bt, g1W2, g1b2, last=False)
    x = gin_layer(x, g2W1, g2b1, g2g, g2bt, g2W2, g2b2, last=True)

    gemb = jnp.mean(x, axis=0, keepdims=True)
    m = jnp.maximum(_bdot(vm[:], meW1[:]) + meb1[:], 0.0)
    memb = _bdot(m, meW2[:]) + meb2[:]

    w1 = dW1[:]
    row = (_bdot(gemb, w1[H:2 * H, :]) + _bdot(memb, w1[2 * H:, :])
           + db1[:])
    s = jnp.maximum(_bdot(x, w1[:H, :]) + row, 0.0)
    s = jnp.maximum(_bdot(s, dW2[:]) + db2[:], 0.0)
    scores = _bdot(s, dW3[:]) + db3[:]

    scores = jnp.where(ready[:] == 0.0, -jnp.inf, scores)
    mx = jnp.max(scores, axis=0, keepdims=True)
    e = jnp.exp(scores - mx)
    probs_out[:] = e / jnp.sum(e, axis=0, keepdims=True)
    graph_out[:] = gemb
    mach_out[:] = memb


def kernel(task_state_scheduled, task_state_ready, task_completion_time,
           vm_completion_time, adj,
           gin0_W1, gin0_b1, gin0_gamma, gin0_beta, gin0_W2, gin0_b2,
           gin1_W1, gin1_b1, gin1_gamma, gin1_beta, gin1_W2, gin1_b2,
           gin2_W1, gin2_b1, gin2_gamma, gin2_beta, gin2_W2, gin2_b2,
           me_W1, me_b1, me_W2, me_b2,
           dec_W1, dec_b1, dec_W2, dec_b2, dec_W3, dec_b3):
    feats = jnp.stack([task_state_scheduled, task_state_ready,
                       task_completion_time], axis=-1)
    ready = task_state_ready.reshape(N_JOBS, 1)
    vm = vm_completion_time.reshape(1, N_MACHINES)

    def r(v):
        return v.reshape(1, -1)

    probs, gemb, memb = pl.pallas_call(
        _fused_forward,
        out_shape=(
            jax.ShapeDtypeStruct((N_JOBS, 1), _F32),
            jax.ShapeDtypeStruct((1, H), _F32),
            jax.ShapeDtypeStruct((1, H), _F32),
        ),
    )(feats, ready, vm, adj,
      gin0_W1, r(gin0_b1), r(gin0_gamma), r(gin0_beta), gin0_W2, r(gin0_b2),
      gin1_W1, r(gin1_b1), r(gin1_gamma), r(gin1_beta), gin1_W2, r(gin1_b2),
      gin2_W1, r(gin2_b1), r(gin2_gamma), r(gin2_beta), gin2_W2, r(gin2_b2),
      me_W1, r(me_b1), me_W2, r(me_b2),
      dec_W1, r(dec_b1), dec_W2, r(dec_b2), dec_W3, r(dec_b3))

    return (probs.reshape(N_JOBS), gemb, memb)
